# baseline (speedup 1.0000x reference)
import functools

import jax
import jax.numpy as jnp
from jax import lax
from jax.experimental import pallas as pl
from jax.experimental.pallas import tpu as pltpu
from jax.experimental.pallas import tpu_sc as plsc

_D = 64
_CHUNK = 128
_NBUF = 4
_NC = 2
_NS = 16
_NW = _NC * _NS
_LANES = 16


def _body(x_hbm, table_hbm, out_hbm, idx_v, *rest, cpw):
    bufs = rest[:_NBUF]
    gsems = rest[_NBUF:2 * _NBUF]
    ssems = rest[2 * _NBUF:3 * _NBUF]

    wid = lax.axis_index("s") * _NC + lax.axis_index("c")
    c0 = wid * cpw

    pltpu.sync_copy(x_hbm.at[pl.ds(c0, cpw)], idx_v)

    def fire_gather(b, g):
        pltpu.async_copy(table_hbm.at[idx_v.at[g]], bufs[b], gsems[b])

    def wait_gather(b, g):
        pltpu.make_async_copy(table_hbm.at[idx_v.at[g]], bufs[b], gsems[b]).wait()

    def fire_store(b, g):
        pltpu.async_copy(bufs[b], out_hbm.at[pl.ds((c0 + g) * _CHUNK, _CHUNK)],
                         ssems[b])

    def wait_store(b, g):
        pltpu.make_async_copy(bufs[b],
                              out_hbm.at[pl.ds((c0 + g) * _CHUNK, _CHUNK)],
                              ssems[b]).wait()

    def fix_padding(b, g):
        buf = bufs[b]
        zeros = jnp.zeros((_LANES,), jnp.float32)
        for v in range(_CHUNK // _LANES):
            ivec = idx_v[g, pl.ds(v * _LANES, _LANES)]
            nzero = plsc.all_reduce_population_count(ivec == 0)

            @pl.when(nzero[0] > 0)
            def _():
                rows = lax.iota(jnp.int32, _LANES) + (v * _LANES)
                msk = ivec == 0

                def zcol(j, carry):
                    cols = jnp.zeros((_LANES,), jnp.int32) + j
                    plsc.store_scatter(buf, [rows, cols], zeros, mask=msk)
                    return carry

                lax.fori_loop(0, _D, zcol, 0)

    for b in range(_NBUF):
        fire_gather(b, b)

    def outer(i, carry):
        for b in range(_NBUF):
            g = i * _NBUF + b
            wait_gather(b, g)
            fix_padding(b, g)
            fire_store(b, g)

            @pl.when(g + _NBUF < cpw)
            def _():
                wait_store(b, g)
                fire_gather(b, g + _NBUF)
        return carry

    lax.fori_loop(0, cpw // _NBUF, outer, 0)

    for b in range(_NBUF):
        wait_store(b, cpw - _NBUF + b)


def kernel(X, table):
    batch, hist = X.shape
    rows = batch * hist
    n_chunks = rows // _CHUNK
    cpw = n_chunks // _NW

    x2d = X.reshape(n_chunks, _CHUNK)
    mesh = plsc.VectorSubcoreMesh(core_axis_name="c", subcore_axis_name="s",
                                  num_cores=_NC, num_subcores=_NS)
    scratch = (
        [pltpu.VMEM((cpw, _CHUNK), jnp.int32)]
        + [pltpu.VMEM((_CHUNK, _D), jnp.float32)] * _NBUF
        + [pltpu.SemaphoreType.DMA] * (2 * _NBUF)
    )
    out = pl.kernel(
        functools.partial(_body, cpw=cpw),
        out_type=jax.ShapeDtypeStruct((rows, _D), jnp.float32),
        mesh=mesh,
        scratch_types=scratch,
        compiler_params=pltpu.CompilerParams(needs_layout_passes=False,
                                             use_tc_tiling_on_sc=False),
    )(x2d, table)
    return out.reshape(batch, hist, _D)
